# baseline (speedup 1.0000x reference)
import jax
import jax.numpy as jnp
from jax.experimental import pallas as pl

N = 10000


def _bn(x, g, b):
    m = jnp.mean(x, axis=0)
    v = jnp.var(x, axis=0)
    return (x - m) / jnp.sqrt(v + 1e-5) * g + b


def _mlp(x, Wa, ba, ga, bea, Wb, bb, gb, beb):
    x = jax.nn.relu(_bn(x @ Wa + ba, ga, bea))
    return _bn(x @ Wb + bb, gb, beb)


def kernel(in_feat, edge_index, W1a, b1a, g1a, be1a, W1b, b1b, g1b, be1b,
           W2a, b2a, g2a, be2a, W2b, b2b, g2b, be2b):
    src = edge_index[0]
    dst = edge_index[1]

    def gin_conv(x, Wa, ba, ga, bea, Wb, bb, gb, beb):
        agg = jax.ops.segment_sum(x[src], dst, num_segments=N)
        return _mlp(x + agg, Wa, ba, ga, bea, Wb, bb, gb, beb)

    h = jax.nn.relu(gin_conv(in_feat, W1a, b1a, g1a, be1a, W1b, b1b, g1b, be1b))
    h = gin_conv(h, W2a, b2a, g2a, be2a, W2b, b2b, g2b, be2b)
    return jnp.mean(h, axis=0, keepdims=True)


# speedup vs baseline: 2.6756x; 2.6756x over previous
import functools

import jax
import jax.numpy as jnp
import numpy as np
from jax import lax
from jax.experimental import pallas as pl
from jax.experimental.pallas import tpu as pltpu
from jax.experimental.pallas import tpu_sc as plsc

N = 10000
E = 320000
D = 128
H = 128
C = 40

NPAD = 10016
EPAD = 320128
WIN = 240

def _range_bounds():
    bs = []
    for s in range(2):
        for t in range(16):
            bs.append(160000 * s + min(240 * (41 * t + min(t, 11)), 160000))
    bs.append(E)
    while len(bs) < 40:
        bs.append(E)
    return bs

_BOUNDS = _range_bounds()


def _sc_segment_sum(feats, dsts_pad, srcs_pad):
    mesh = plsc.VectorSubcoreMesh(core_axis_name="c", subcore_axis_name="s")

    @functools.partial(
        pl.kernel, mesh=mesh,
        out_type=[
            jax.ShapeDtypeStruct((NPAD * 128,), jnp.float32),
            jax.ShapeDtypeStruct((32, 16), jnp.int32),
            jax.ShapeDtypeStruct((64 * 128,), jnp.float32),
        ],
        scratch_types=[
            pltpu.VMEM((WIN,), jnp.int32),
            pltpu.VMEM((WIN + 16,), jnp.int32),
            pltpu.VMEM((WIN, 128), jnp.float32),
            pltpu.VMEM((4 * 128,), jnp.float32),
            pltpu.VMEM((128,), jnp.float32),
            pltpu.VMEM((16,), jnp.int32),
            pltpu.VMEM((16,), jnp.int32),
            pltpu.SemaphoreType.DMA,
            pltpu.SemaphoreType.DMA,
            pltpu.SemaphoreType.DMA,
            pltpu.SemaphoreType.DMA,
            pltpu.SemaphoreType.DMA,
            pltpu.SemaphoreType.DMA,
        ],
    )
    def scat(feats_h, dsts_h, srcs_h, agg_h, bids_h, brows_h,
             idxv, dstv, rowsv, ringv, zerov, idsv, nfv,
             gsem, fs0, fs1, fs2, fs3, msem):
        wid = lax.axis_index("s") * 2 + lax.axis_index("c")
        def bound_of(w):
            s = w // 16
            t = w - 16 * s
            raw = 240 * (41 * t + jnp.minimum(t, 11))
            return 160000 * s + jnp.minimum(raw, 160000)
        b_lo = jnp.where(wid >= 32, jnp.int32(E), bound_of(wid))
        b_hi = jnp.where(wid + 1 >= 32, jnp.int32(E), bound_of(wid + 1))
        cnt = b_hi - b_lo
        nwin = (cnt + WIN - 1) // WIN

        fsems = (fs0, fs1, fs2, fs3)

        zerov[...] = jnp.zeros((128,), jnp.float32)

        def flush(prev, ord_, acc):
            slot = lax.rem(ord_, 4)
            def wait_slot(s):
                def w(_):
                    pltpu.make_async_copy(
                        ringv.at[pl.ds(s * 128, 128)],
                        agg_h.at[pl.ds(0, 128)],
                        fsems[s]).wait()
                    return 0
                return w
            for s in range(4):
                _ = lax.cond((slot == s) & (ord_ >= 4), wait_slot(s),
                             lambda _: 0, 0)
            for k in range(8):
                ringv[pl.ds(slot * 128 + k * 16, 16)] = acc[k]
            def to_boundary(_):
                for s in range(4):
                    _ = lax.cond(slot == s, lambda _: (pltpu.async_copy(
                        ringv.at[pl.ds(s * 128, 128)],
                        brows_h.at[pl.ds((2 * wid) * 128, 128)],
                        fsems[s]), 0)[1], lambda _: 0, 0)
                return 0
            def to_direct(_):
                for s in range(4):
                    _ = lax.cond(slot == s, lambda _: (pltpu.async_copy(
                        ringv.at[pl.ds(s * 128, 128)],
                        agg_h.at[pl.ds(prev * 128, 128)],
                        fsems[s]), 0)[1], lambda _: 0, 0)
                return 0
            _ = lax.cond(ord_ == 0, to_boundary, to_direct, 0)
            return 0

        def win_body(jw, carry):
            (prev, ord_, id0) = carry[0]
            acc = carry[1]
            wstart = b_lo + jw * WIN
            trip = lax.min(jnp.int32(WIN), cnt - jw * WIN)
            pltpu.sync_copy(srcs_h.at[pl.ds(wstart, WIN)], idxv)
            pltpu.sync_copy(dsts_h.at[pl.ds(wstart, WIN)],
                            dstv.at[pl.ds(0, WIN)])
            pltpu.async_copy(feats_h.at[idxv], rowsv, gsem).wait()

            def edge_body(i, ec):
                (prev, ord_, id0) = ec[0]
                acc = ec[1]
                n = dstv[pl.ds(i, 16)][0]
                r = tuple(rowsv[i, pl.ds(k * 16, 16)] for k in range(8))
                is_new = n != prev
                do_flush = is_new & (prev >= 0)
                def yes(_):
                    flush(prev, ord_, acc)
                    return 0
                _ = lax.cond(do_flush, yes, lambda _: 0, 0)
                id0n = jnp.where(do_flush & (ord_ == 0), prev, id0)
                ordn = jnp.where(do_flush, ord_ + 1, ord_)
                accn = tuple(
                    jnp.where(is_new, r[k], acc[k] + r[k]) for k in range(8))
                return ((n, ordn, id0n), accn)

            ec = lax.fori_loop(0, trip, edge_body, ((prev, ord_, id0), acc))
            return ec

        zero8 = tuple(jnp.zeros((16,), jnp.float32) for _ in range(8))
        carry = lax.fori_loop(
            0, nwin, win_body,
            ((jnp.int32(-1), jnp.int32(0), jnp.int32(-1)), zero8))
        (prev, ord_, id0) = carry[0]
        acc = carry[1]

        for k in range(8):
            ringv[pl.ds(k * 16, 16)] = acc[k]
        def carry_first(_):
            pltpu.async_copy(ringv.at[pl.ds(0, 128)],
                             brows_h.at[pl.ds((2 * wid) * 128, 128)],
                             msem).wait()
            return 0
        def carry_second(_):
            pltpu.async_copy(ringv.at[pl.ds(0, 128)],
                             brows_h.at[pl.ds((2 * wid + 1) * 128, 128)],
                             msem).wait()
            return 0
        _ = lax.cond(ord_ == 0, carry_first, carry_second, 0)
        id0f = jnp.where(ord_ == 0, prev, id0)
        id1f = jnp.where(ord_ == 0, jnp.int32(-1), prev)

        lane = lax.iota(jnp.int32, 16)
        idsv[...] = jnp.where(lane == 0, id0f,
                              jnp.where(lane == 1, id1f, jnp.int32(-1)))
        pltpu.sync_copy(idsv, bids_h.at[wid])

        for s in range(4):
            def dr(_):
                pltpu.make_async_copy(
                    ringv.at[pl.ds(s * 128, 128)],
                    agg_h.at[pl.ds(0, 128)], fsems[s]).wait()
                return 0
            _ = lax.cond(ord_ > s, dr, lambda _: 0, 0)

        def nf_read(_):
            pltpu.sync_copy(dsts_h.at[pl.ds(b_hi, 16)], nfv)
            return nfv[...][0]
        next_first = lax.cond(wid == 31, lambda _: jnp.int32(NPAD - 1),
                              nf_read, 0)
        zlo = prev
        zhi = next_first
        def zrow(j, _):
            pltpu.sync_copy(zerov, agg_h.at[pl.ds((zlo + j) * 128, 128)])
            return 0
        _ = lax.fori_loop(0, zhi - zlo + 1, zrow, 0)
        def w0_extra(_):
            pltpu.sync_copy(dsts_h.at[pl.ds(0, 16)], nfv)
            first0 = nfv[...][0]
            def zr(j, _):
                pltpu.sync_copy(zerov, agg_h.at[pl.ds(j * 128, 128)])
                return 0
            _ = lax.fori_loop(0, first0 + 1, zr, 0)
            return 0
        _ = lax.cond(wid == 0, w0_extra, lambda _: 0, 0)

    return scat(feats, dsts_pad, srcs_pad)


def _strip_reduce128(ref, fn, sizes):
    total = jnp.zeros((1, 128), jnp.float32)
    pos = 0
    for sz in sizes:
        base = pos
        def body(i, acc):
            return acc + fn(ref[pl.ds((base + i) * 8, 8), :])
        acc = lax.fori_loop(0, sz, body, jnp.zeros((8, 128), jnp.float32))
        pos += sz
        a4 = acc[:4] + acc[4:]
        a2 = a4[:2] + a4[2:]
        total = total + (a2[0:1] + a2[1:2])
    return total


def _lane_reduce40(ref, fn):
    def body(i, acc):
        return acc + fn(ref[pl.ds(i * 128, 128), :])
    acc = lax.fori_loop(0, 78, body, jnp.zeros((128, C), jnp.float32))
    tailblk = fn(ref[pl.ds(9984, 16), :])
    acc = acc + jnp.pad(tailblk, ((0, 112), (0, 0)))
    b = acc[0:8]
    for g in range(1, 16):
        b = b + acc[8 * g:8 * g + 8]
    b4 = b[:4] + b[4:]
    b2 = b4[:2] + b4[2:]
    return b2[0:1] + b2[1:2]


def _merge_boundary(aggm_ref, bids_ref, brows_ref):
    iota8 = lax.broadcasted_iota(jnp.int32, (8, 128), 0)
    def body(e, _):
        w = e // 2
        k = e - 2 * w
        nid = bids_ref[w, k]
        valid = nid >= 0
        nidc = jnp.where(valid, nid, 0)
        n8 = pl.multiple_of((nidc // 8) * 8, 8)
        roff = nidc - n8
        block = aggm_ref[pl.ds(n8, 8), :]
        row = brows_ref[pl.ds(e, 1), :]
        rowb = jnp.broadcast_to(row, (8, 128))
        mask = (iota8 == roff) & valid
        aggm_ref[pl.ds(n8, 8), :] = jnp.where(mask, block + rowb, block)
        return 0
    lax.fori_loop(0, 64, body, 0)


def _bn_chain128(z_ref, g, b):
    m = _strip_reduce128(z_ref, lambda u: u, (1250,)) * np.float32(1e-4)
    v = _strip_reduce128(z_ref, lambda u: (u - m) * (u - m),
                         (625, 625)) * np.float32(1e-4)
    sd = jnp.sqrt(v + np.float32(1e-5))
    return (z_ref[...] - m) / sd * g + b


def _tc_layer1(x, agg, bids, brows, W1a, b1a, g1a, be1a, W1b, b1b, g1b, be1b):
    def kern(x_ref, agg_ref, bids_ref, brows_ref,
             w1a_ref, b1a_ref, g1a_ref, be1a_ref,
             w1b_ref, b1b_ref, g1b_ref, be1b_ref,
             out_ref, aggm_ref, z_ref, t_ref):
        aggm_ref[...] = agg_ref[...]
        _merge_boundary(aggm_ref, bids_ref, brows_ref)
        t_ref[...] = x_ref[...] + aggm_ref[...]
        z_ref[...] = jnp.dot(t_ref[...], w1a_ref[...],
                             preferred_element_type=jnp.float32) + b1a_ref[...]
        h = _bn_chain128(z_ref, g1a_ref[...], be1a_ref[...])
        t_ref[...] = jnp.maximum(h, 0.0)
        z_ref[...] = jnp.dot(t_ref[...], w1b_ref[...],
                             preferred_element_type=jnp.float32) + b1b_ref[...]
        h2 = _bn_chain128(z_ref, g1b_ref[...], be1b_ref[...])
        out_ref[...] = jnp.maximum(h2, 0.0)

    vspec = pl.BlockSpec(memory_space=pltpu.VMEM)
    specs = [pl.BlockSpec(memory_space=pltpu.SMEM) if i == 2 else vspec
             for i in range(12)]
    return pl.pallas_call(
        kern,
        out_shape=jax.ShapeDtypeStruct((N, 128), jnp.float32),
        in_specs=specs,
        scratch_shapes=[
            pltpu.VMEM((N, 128), jnp.float32),
            pltpu.VMEM((N, 128), jnp.float32),
            pltpu.VMEM((N, 128), jnp.float32),
        ],
    )(x, agg, bids, brows,
      W1a, b1a.reshape(1, H), g1a.reshape(1, H), be1a.reshape(1, H),
      W1b, b1b.reshape(1, H), g1b.reshape(1, H), be1b.reshape(1, H))


def _tc_layer2(h1, agg, bids, brows, W2a, b2a, g2a, be2a, W2b, b2b, g2b, be2b):
    def kern(x_ref, agg_ref, bids_ref, brows_ref,
             w2a_ref, b2a_ref, g2a_ref, be2a_ref,
             w2b_ref, b2b_ref, g2b_ref, be2b_ref,
             out_ref, aggm_ref, z_ref, t_ref, z40_ref):
        aggm_ref[...] = agg_ref[...]
        _merge_boundary(aggm_ref, bids_ref, brows_ref)
        t_ref[...] = x_ref[...] + aggm_ref[...]
        z_ref[...] = jnp.dot(t_ref[...], w2a_ref[...],
                             preferred_element_type=jnp.float32) + b2a_ref[...]
        h = _bn_chain128(z_ref, g2a_ref[...], be2a_ref[...])
        t_ref[...] = jnp.maximum(h, 0.0)
        z40_ref[...] = jnp.dot(t_ref[...], w2b_ref[...],
                               preferred_element_type=jnp.float32) + b2b_ref[...]
        m40 = _lane_reduce40(z40_ref, lambda u: u) * np.float32(1e-4)
        v40 = _lane_reduce40(z40_ref, lambda u: (u - m40) * (u - m40)) \
            * np.float32(1e-4)
        sd40 = jnp.sqrt(v40 + np.float32(1e-5))
        g40 = g2b_ref[...]
        be40 = be2b_ref[...]
        fin = _lane_reduce40(
            z40_ref, lambda u: (u - m40) / sd40 * g40 + be40) * np.float32(1e-4)
        out_ref[...] = fin

    vspec = pl.BlockSpec(memory_space=pltpu.VMEM)
    specs = [pl.BlockSpec(memory_space=pltpu.SMEM) if i == 2 else vspec
             for i in range(12)]
    return pl.pallas_call(
        kern,
        out_shape=jax.ShapeDtypeStruct((1, C), jnp.float32),
        in_specs=specs,
        scratch_shapes=[
            pltpu.VMEM((N, 128), jnp.float32),
            pltpu.VMEM((N, 128), jnp.float32),
            pltpu.VMEM((N, 128), jnp.float32),
            pltpu.VMEM((N, C), jnp.float32),
        ],
    )(h1, agg, bids, brows,
      W2a, b2a.reshape(1, H), g2a.reshape(1, H), be2a.reshape(1, H),
      W2b, b2b.reshape(1, C), g2b.reshape(1, C), be2b.reshape(1, C))


def kernel(in_feat, edge_index, W1a, b1a, g1a, be1a, W1b, b1b, g1b, be1b,
           W2a, b2a, g2a, be2a, W2b, b2b, g2b, be2b):
    src = edge_index[0]
    dst = edge_index[1]
    dst_s, src_s = lax.sort((dst, src), dimension=0, is_stable=True,
                            num_keys=1)
    pad = jnp.zeros((EPAD - E,), jnp.int32)
    dsts_pad = jnp.concatenate([dst_s, pad])
    srcs_pad = jnp.concatenate([src_s, pad])

    agg1d, bids, brows1d = _sc_segment_sum(in_feat, dsts_pad, srcs_pad)
    agg = agg1d.reshape(NPAD, 128)[:N]
    brows = brows1d.reshape(64, 128)
    h1 = _tc_layer1(in_feat, agg, bids, brows,
                    W1a, b1a, g1a, be1a, W1b, b1b, g1b, be1b)

    agg1d2, bids2, brows1d2 = _sc_segment_sum(h1, dsts_pad, srcs_pad)
    agg2 = agg1d2.reshape(NPAD, 128)[:N]
    brows2 = brows1d2.reshape(64, 128)
    out = _tc_layer2(h1, agg2, bids2, brows2,
                     W2a, b2a, g2a, be2a, W2b, b2b, g2b, be2b)
    return out


# speedup vs baseline: 2.9320x; 1.0958x over previous
import functools

import jax
import jax.numpy as jnp
import numpy as np
from jax import lax
from jax.experimental import pallas as pl
from jax.experimental.pallas import tpu as pltpu
from jax.experimental.pallas import tpu_sc as plsc

N = 10000
E = 320000
D = 128
H = 128
C = 40

NPAD = 10016
EPAD = 320128
WIN = 240

def _range_bounds():
    bs = []
    for s in range(2):
        for t in range(16):
            bs.append(160000 * s + min(240 * (41 * t + min(t, 11)), 160000))
    bs.append(E)
    while len(bs) < 40:
        bs.append(E)
    return bs

_BOUNDS = _range_bounds()


def _sc_segment_sum(feats, dsts_pad, srcs_pad):
    mesh = plsc.VectorSubcoreMesh(core_axis_name="c", subcore_axis_name="s")

    @functools.partial(
        pl.kernel, mesh=mesh,
        out_type=[
            jax.ShapeDtypeStruct((NPAD * 128,), jnp.float32),
            jax.ShapeDtypeStruct((32, 16), jnp.int32),
            jax.ShapeDtypeStruct((64 * 128,), jnp.float32),
        ],
        scratch_types=[
            pltpu.VMEM((WIN,), jnp.int32),
            pltpu.VMEM((WIN + 16,), jnp.int32),
            pltpu.VMEM((WIN, 128), jnp.float32),
            pltpu.VMEM((4 * 128,), jnp.float32),
            pltpu.VMEM((128,), jnp.float32),
            pltpu.VMEM((16,), jnp.int32),
            pltpu.VMEM((16,), jnp.int32),
            pltpu.SemaphoreType.DMA,
            pltpu.SemaphoreType.DMA,
            pltpu.SemaphoreType.DMA,
            pltpu.SemaphoreType.DMA,
            pltpu.SemaphoreType.DMA,
            pltpu.SemaphoreType.DMA,
        ],
    )
    def scat(feats_h, dsts_h, srcs_h, agg_h, bids_h, brows_h,
             idxv, dstv, rowsv, ringv, zerov, idsv, nfv,
             gsem, fs0, fs1, fs2, fs3, msem):
        wid = lax.axis_index("s") * 2 + lax.axis_index("c")
        def bound_of(w):
            s = w // 16
            t = w - 16 * s
            raw = 240 * (41 * t + jnp.minimum(t, 11))
            return 160000 * s + jnp.minimum(raw, 160000)
        b_lo = jnp.where(wid >= 32, jnp.int32(E), bound_of(wid))
        b_hi = jnp.where(wid + 1 >= 32, jnp.int32(E), bound_of(wid + 1))
        cnt = b_hi - b_lo
        nwin = (cnt + WIN - 1) // WIN

        fsems = (fs0, fs1, fs2, fs3)

        zerov[...] = jnp.zeros((128,), jnp.float32)

        def flush(prev, ord_, acc):
            slot = lax.rem(ord_, 4)
            def wait_slot(s):
                def w(_):
                    pltpu.make_async_copy(
                        ringv.at[pl.ds(s * 128, 128)],
                        agg_h.at[pl.ds(0, 128)],
                        fsems[s]).wait()
                    return 0
                return w
            for s in range(4):
                _ = lax.cond((slot == s) & (ord_ >= 4), wait_slot(s),
                             lambda _: 0, 0)
            for k in range(8):
                ringv[pl.ds(slot * 128 + k * 16, 16)] = acc[k]
            def to_boundary(_):
                for s in range(4):
                    _ = lax.cond(slot == s, lambda _: (pltpu.async_copy(
                        ringv.at[pl.ds(s * 128, 128)],
                        brows_h.at[pl.ds((2 * wid) * 128, 128)],
                        fsems[s]), 0)[1], lambda _: 0, 0)
                return 0
            def to_direct(_):
                for s in range(4):
                    _ = lax.cond(slot == s, lambda _: (pltpu.async_copy(
                        ringv.at[pl.ds(s * 128, 128)],
                        agg_h.at[pl.ds(prev * 128, 128)],
                        fsems[s]), 0)[1], lambda _: 0, 0)
                return 0
            _ = lax.cond(ord_ == 0, to_boundary, to_direct, 0)
            return 0

        def win_body(jw, carry):
            (prev, ord_, id0) = carry[0]
            acc = carry[1]
            wstart = b_lo + jw * WIN
            trip = lax.min(jnp.int32(WIN), cnt - jw * WIN)
            pltpu.sync_copy(srcs_h.at[pl.ds(wstart, WIN)], idxv)
            pltpu.sync_copy(dsts_h.at[pl.ds(wstart, WIN)],
                            dstv.at[pl.ds(0, WIN)])
            pltpu.async_copy(feats_h.at[idxv], rowsv, gsem).wait()

            def grp_body(g, ec):
                (prev, ord_, id0) = ec[0]
                acc = ec[1]
                dvec = dstv[pl.ds(g * 16, 16)]
                for j in range(16):
                    n = dvec[j]
                    r = tuple(rowsv[g * 16 + j, pl.ds(k * 16, 16)]
                              for k in range(8))
                    is_new = n != prev
                    do_flush = is_new & (prev >= 0)
                    def yes(_, prev=prev, ord_=ord_, acc=acc):
                        flush(prev, ord_, acc)
                        return 0
                    _ = lax.cond(do_flush, yes, lambda _: 0, 0)
                    id0 = jnp.where(do_flush & (ord_ == 0), prev, id0)
                    ord_ = jnp.where(do_flush, ord_ + 1, ord_)
                    acc = tuple(
                        jnp.where(is_new, r[k], acc[k] + r[k])
                        for k in range(8))
                    prev = n
                return ((prev, ord_, id0), acc)

            ec = lax.fori_loop(0, trip // 16, grp_body,
                              ((prev, ord_, id0), acc))
            return ec

        zero8 = tuple(jnp.zeros((16,), jnp.float32) for _ in range(8))
        carry = lax.fori_loop(
            0, nwin, win_body,
            ((jnp.int32(-1), jnp.int32(0), jnp.int32(-1)), zero8))
        (prev, ord_, id0) = carry[0]
        acc = carry[1]

        for k in range(8):
            ringv[pl.ds(k * 16, 16)] = acc[k]
        def carry_first(_):
            pltpu.async_copy(ringv.at[pl.ds(0, 128)],
                             brows_h.at[pl.ds((2 * wid) * 128, 128)],
                             msem).wait()
            return 0
        def carry_second(_):
            pltpu.async_copy(ringv.at[pl.ds(0, 128)],
                             brows_h.at[pl.ds((2 * wid + 1) * 128, 128)],
                             msem).wait()
            return 0
        _ = lax.cond(ord_ == 0, carry_first, carry_second, 0)
        id0f = jnp.where(ord_ == 0, prev, id0)
        id1f = jnp.where(ord_ == 0, jnp.int32(-1), prev)

        lane = lax.iota(jnp.int32, 16)
        idsv[...] = jnp.where(lane == 0, id0f,
                              jnp.where(lane == 1, id1f, jnp.int32(-1)))
        pltpu.sync_copy(idsv, bids_h.at[wid])

        for s in range(4):
            def dr(_):
                pltpu.make_async_copy(
                    ringv.at[pl.ds(s * 128, 128)],
                    agg_h.at[pl.ds(0, 128)], fsems[s]).wait()
                return 0
            _ = lax.cond(ord_ > s, dr, lambda _: 0, 0)

        def nf_read(_):
            pltpu.sync_copy(dsts_h.at[pl.ds(b_hi, 16)], nfv)
            return nfv[...][0]
        next_first = lax.cond(wid == 31, lambda _: jnp.int32(NPAD - 1),
                              nf_read, 0)
        zlo = prev
        zhi = next_first
        def zrow(j, _):
            pltpu.sync_copy(zerov, agg_h.at[pl.ds((zlo + j) * 128, 128)])
            return 0
        _ = lax.fori_loop(0, zhi - zlo + 1, zrow, 0)
        def w0_extra(_):
            pltpu.sync_copy(dsts_h.at[pl.ds(0, 16)], nfv)
            first0 = nfv[...][0]
            def zr(j, _):
                pltpu.sync_copy(zerov, agg_h.at[pl.ds(j * 128, 128)])
                return 0
            _ = lax.fori_loop(0, first0 + 1, zr, 0)
            return 0
        _ = lax.cond(wid == 0, w0_extra, lambda _: 0, 0)

    return scat(feats, dsts_pad, srcs_pad)


def _strip_reduce128(ref, fn, sizes):
    total = jnp.zeros((1, 128), jnp.float32)
    pos = 0
    for sz in sizes:
        base = pos
        def body(i, acc):
            return acc + fn(ref[pl.ds((base + i) * 8, 8), :])
        acc = lax.fori_loop(0, sz, body, jnp.zeros((8, 128), jnp.float32))
        pos += sz
        a4 = acc[:4] + acc[4:]
        a2 = a4[:2] + a4[2:]
        total = total + (a2[0:1] + a2[1:2])
    return total


def _lane_reduce40(ref, fn):
    def body(i, acc):
        return acc + fn(ref[pl.ds(i * 128, 128), :])
    acc = lax.fori_loop(0, 78, body, jnp.zeros((128, C), jnp.float32))
    tailblk = fn(ref[pl.ds(9984, 16), :])
    acc = acc + jnp.pad(tailblk, ((0, 112), (0, 0)))
    b = acc[0:8]
    for g in range(1, 16):
        b = b + acc[8 * g:8 * g + 8]
    b4 = b[:4] + b[4:]
    b2 = b4[:2] + b4[2:]
    return b2[0:1] + b2[1:2]


def _merge_boundary(aggm_ref, bids_ref, brows_ref):
    iota8 = lax.broadcasted_iota(jnp.int32, (8, 128), 0)
    def body(e, _):
        w = e // 2
        k = e - 2 * w
        nid = bids_ref[w, k]
        valid = nid >= 0
        nidc = jnp.where(valid, nid, 0)
        n8 = pl.multiple_of((nidc // 8) * 8, 8)
        roff = nidc - n8
        block = aggm_ref[pl.ds(n8, 8), :]
        row = brows_ref[pl.ds(e, 1), :]
        rowb = jnp.broadcast_to(row, (8, 128))
        mask = (iota8 == roff) & valid
        aggm_ref[pl.ds(n8, 8), :] = jnp.where(mask, block + rowb, block)
        return 0
    lax.fori_loop(0, 64, body, 0)


def _bn_chain128(z_ref, g, b):
    m = _strip_reduce128(z_ref, lambda u: u, (1250,)) * np.float32(1e-4)
    v = _strip_reduce128(z_ref, lambda u: (u - m) * (u - m),
                         (625, 625)) * np.float32(1e-4)
    sd = jnp.sqrt(v + np.float32(1e-5))
    return (z_ref[...] - m) / sd * g + b


def _tc_layer1(x, agg, bids, brows, W1a, b1a, g1a, be1a, W1b, b1b, g1b, be1b):
    def kern(x_ref, agg_ref, bids_ref, brows_ref,
             w1a_ref, b1a_ref, g1a_ref, be1a_ref,
             w1b_ref, b1b_ref, g1b_ref, be1b_ref,
             out_ref, aggm_ref, z_ref, t_ref):
        aggm_ref[...] = agg_ref[...]
        _merge_boundary(aggm_ref, bids_ref, brows_ref)
        t_ref[...] = x_ref[...] + aggm_ref[...]
        z_ref[...] = jnp.dot(t_ref[...], w1a_ref[...],
                             preferred_element_type=jnp.float32) + b1a_ref[...]
        h = _bn_chain128(z_ref, g1a_ref[...], be1a_ref[...])
        t_ref[...] = jnp.maximum(h, 0.0)
        z_ref[...] = jnp.dot(t_ref[...], w1b_ref[...],
                             preferred_element_type=jnp.float32) + b1b_ref[...]
        h2 = _bn_chain128(z_ref, g1b_ref[...], be1b_ref[...])
        out_ref[...] = jnp.maximum(h2, 0.0)

    vspec = pl.BlockSpec(memory_space=pltpu.VMEM)
    specs = [pl.BlockSpec(memory_space=pltpu.SMEM) if i == 2 else vspec
             for i in range(12)]
    return pl.pallas_call(
        kern,
        out_shape=jax.ShapeDtypeStruct((N, 128), jnp.float32),
        in_specs=specs,
        scratch_shapes=[
            pltpu.VMEM((N, 128), jnp.float32),
            pltpu.VMEM((N, 128), jnp.float32),
            pltpu.VMEM((N, 128), jnp.float32),
        ],
    )(x, agg, bids, brows,
      W1a, b1a.reshape(1, H), g1a.reshape(1, H), be1a.reshape(1, H),
      W1b, b1b.reshape(1, H), g1b.reshape(1, H), be1b.reshape(1, H))


def _tc_layer2(h1, agg, bids, brows, W2a, b2a, g2a, be2a, W2b, b2b, g2b, be2b):
    def kern(x_ref, agg_ref, bids_ref, brows_ref,
             w2a_ref, b2a_ref, g2a_ref, be2a_ref,
             w2b_ref, b2b_ref, g2b_ref, be2b_ref,
             out_ref, aggm_ref, z_ref, t_ref, z40_ref):
        aggm_ref[...] = agg_ref[...]
        _merge_boundary(aggm_ref, bids_ref, brows_ref)
        t_ref[...] = x_ref[...] + aggm_ref[...]
        z_ref[...] = jnp.dot(t_ref[...], w2a_ref[...],
                             preferred_element_type=jnp.float32) + b2a_ref[...]
        h = _bn_chain128(z_ref, g2a_ref[...], be2a_ref[...])
        t_ref[...] = jnp.maximum(h, 0.0)
        z40_ref[...] = jnp.dot(t_ref[...], w2b_ref[...],
                               preferred_element_type=jnp.float32) + b2b_ref[...]
        m40 = _lane_reduce40(z40_ref, lambda u: u) * np.float32(1e-4)
        v40 = _lane_reduce40(z40_ref, lambda u: (u - m40) * (u - m40)) \
            * np.float32(1e-4)
        sd40 = jnp.sqrt(v40 + np.float32(1e-5))
        g40 = g2b_ref[...]
        be40 = be2b_ref[...]
        fin = _lane_reduce40(
            z40_ref, lambda u: (u - m40) / sd40 * g40 + be40) * np.float32(1e-4)
        out_ref[...] = fin

    vspec = pl.BlockSpec(memory_space=pltpu.VMEM)
    specs = [pl.BlockSpec(memory_space=pltpu.SMEM) if i == 2 else vspec
             for i in range(12)]
    return pl.pallas_call(
        kern,
        out_shape=jax.ShapeDtypeStruct((1, C), jnp.float32),
        in_specs=specs,
        scratch_shapes=[
            pltpu.VMEM((N, 128), jnp.float32),
            pltpu.VMEM((N, 128), jnp.float32),
            pltpu.VMEM((N, 128), jnp.float32),
            pltpu.VMEM((N, C), jnp.float32),
        ],
    )(h1, agg, bids, brows,
      W2a, b2a.reshape(1, H), g2a.reshape(1, H), be2a.reshape(1, H),
      W2b, b2b.reshape(1, C), g2b.reshape(1, C), be2b.reshape(1, C))


def kernel(in_feat, edge_index, W1a, b1a, g1a, be1a, W1b, b1b, g1b, be1b,
           W2a, b2a, g2a, be2a, W2b, b2b, g2b, be2b):
    src = edge_index[0]
    dst = edge_index[1]
    dst_s, src_s = lax.sort((dst, src), dimension=0, is_stable=True,
                            num_keys=1)
    pad = jnp.zeros((EPAD - E,), jnp.int32)
    dsts_pad = jnp.concatenate([dst_s, pad])
    srcs_pad = jnp.concatenate([src_s, pad])

    agg1d, bids, brows1d = _sc_segment_sum(in_feat, dsts_pad, srcs_pad)
    agg = agg1d.reshape(NPAD, 128)[:N]
    brows = brows1d.reshape(64, 128)
    h1 = _tc_layer1(in_feat, agg, bids, brows,
                    W1a, b1a, g1a, be1a, W1b, b1b, g1b, be1b)

    agg1d2, bids2, brows1d2 = _sc_segment_sum(h1, dsts_pad, srcs_pad)
    agg2 = agg1d2.reshape(NPAD, 128)[:N]
    brows2 = brows1d2.reshape(64, 128)
    out = _tc_layer2(h1, agg2, bids2, brows2,
                     W2a, b2a, g2a, be2a, W2b, b2b, g2b, be2b)
    return out


# speedup vs baseline: 2.9329x; 1.0003x over previous
import functools

import jax
import jax.numpy as jnp
import numpy as np
from jax import lax
from jax.experimental import pallas as pl
from jax.experimental.pallas import tpu as pltpu
from jax.experimental.pallas import tpu_sc as plsc

N = 10000
E = 320000
D = 128
H = 128
C = 40

NPAD = 10016
EPAD = 320128
WIN = 240

def _range_bounds():
    bs = []
    for s in range(2):
        for t in range(16):
            bs.append(160000 * s + min(240 * (41 * t + min(t, 11)), 160000))
    bs.append(E)
    while len(bs) < 40:
        bs.append(E)
    return bs

_BOUNDS = _range_bounds()


def _sc_segment_sum(feats, dsts_pad, srcs_pad):
    mesh = plsc.VectorSubcoreMesh(core_axis_name="c", subcore_axis_name="s")

    @functools.partial(
        pl.kernel, mesh=mesh,
        out_type=[
            jax.ShapeDtypeStruct((NPAD * 128,), jnp.float32),
            jax.ShapeDtypeStruct((32, 16), jnp.int32),
            jax.ShapeDtypeStruct((64 * 128,), jnp.float32),
        ],
        scratch_types=[
            pltpu.VMEM((WIN,), jnp.int32),
            pltpu.VMEM((WIN,), jnp.int32),
            pltpu.VMEM((WIN + 16,), jnp.int32),
            pltpu.VMEM((WIN + 16,), jnp.int32),
            pltpu.VMEM((WIN, 128), jnp.float32),
            pltpu.VMEM((WIN, 128), jnp.float32),
            pltpu.VMEM((4 * 128,), jnp.float32),
            pltpu.VMEM((128,), jnp.float32),
            pltpu.VMEM((16,), jnp.int32),
            pltpu.VMEM((16,), jnp.int32),
            pltpu.SemaphoreType.DMA,
            pltpu.SemaphoreType.DMA,
            pltpu.SemaphoreType.DMA,
            pltpu.SemaphoreType.DMA,
            pltpu.SemaphoreType.DMA,
            pltpu.SemaphoreType.DMA,
            pltpu.SemaphoreType.DMA,
        ],
    )
    def scat(feats_h, dsts_h, srcs_h, agg_h, bids_h, brows_h,
             idxv0, idxv1, dstv0, dstv1, rowsv0, rowsv1,
             ringv, zerov, idsv, nfv,
             gsem0, gsem1, fs0, fs1, fs2, fs3, msem):
        wid = lax.axis_index("s") * 2 + lax.axis_index("c")
        def bound_of(w):
            s = w // 16
            t = w - 16 * s
            raw = 240 * (41 * t + jnp.minimum(t, 11))
            return 160000 * s + jnp.minimum(raw, 160000)
        b_lo = jnp.where(wid >= 32, jnp.int32(E), bound_of(wid))
        b_hi = jnp.where(wid + 1 >= 32, jnp.int32(E), bound_of(wid + 1))
        cnt = b_hi - b_lo
        nwin = (cnt + WIN - 1) // WIN

        fsems = (fs0, fs1, fs2, fs3)

        zerov[...] = jnp.zeros((128,), jnp.float32)

        def flush(prev, ord_, acc):
            slot = lax.rem(ord_, 4)
            def wait_slot(s):
                def w(_):
                    pltpu.make_async_copy(
                        ringv.at[pl.ds(s * 128, 128)],
                        agg_h.at[pl.ds(0, 128)],
                        fsems[s]).wait()
                    return 0
                return w
            for s in range(4):
                _ = lax.cond((slot == s) & (ord_ >= 4), wait_slot(s),
                             lambda _: 0, 0)
            for k in range(8):
                ringv[pl.ds(slot * 128 + k * 16, 16)] = acc[k]
            def to_boundary(_):
                for s in range(4):
                    _ = lax.cond(slot == s, lambda _: (pltpu.async_copy(
                        ringv.at[pl.ds(s * 128, 128)],
                        brows_h.at[pl.ds((2 * wid) * 128, 128)],
                        fsems[s]), 0)[1], lambda _: 0, 0)
                return 0
            def to_direct(_):
                for s in range(4):
                    _ = lax.cond(slot == s, lambda _: (pltpu.async_copy(
                        ringv.at[pl.ds(s * 128, 128)],
                        agg_h.at[pl.ds(prev * 128, 128)],
                        fsems[s]), 0)[1], lambda _: 0, 0)
                return 0
            _ = lax.cond(ord_ == 0, to_boundary, to_direct, 0)
            return 0

        BUFS = ((idxv0, dstv0, rowsv0, gsem0),
                (idxv1, dstv1, rowsv1, gsem1))

        def prefetch(jw2, bufs):
            idxv_, dstv_, rowsv_, gsem_ = bufs
            ws2 = lax.min(b_lo + jw2 * WIN, jnp.int32(EPAD - WIN))
            pltpu.sync_copy(srcs_h.at[pl.ds(ws2, WIN)], idxv_)
            pltpu.sync_copy(dsts_h.at[pl.ds(ws2, WIN)],
                            dstv_.at[pl.ds(0, WIN)])
            pltpu.async_copy(feats_h.at[idxv_], rowsv_, gsem_)

        def process(jw, ec, bufs):
            idxv_, dstv_, rowsv_, gsem_ = bufs
            (prev, ord_, id0) = ec[0]
            acc = ec[1]
            trip = lax.max(jnp.int32(0),
                           lax.min(jnp.int32(WIN), cnt - jw * WIN))
            pltpu.make_async_copy(feats_h.at[idxv_], rowsv_, gsem_).wait()

            def grp_body(g, ec2):
                (prev, ord_, id0) = ec2[0]
                acc = ec2[1]
                dvec = dstv_[pl.ds(g * 16, 16)]
                for j in range(16):
                    n = dvec[j]
                    r = tuple(rowsv_[g * 16 + j, pl.ds(k * 16, 16)]
                              for k in range(8))
                    is_new = n != prev
                    do_flush = is_new & (prev >= 0)
                    def yes(_, prev=prev, ord_=ord_, acc=acc):
                        flush(prev, ord_, acc)
                        return 0
                    _ = lax.cond(do_flush, yes, lambda _: 0, 0)
                    id0 = jnp.where(do_flush & (ord_ == 0), prev, id0)
                    ord_ = jnp.where(do_flush, ord_ + 1, ord_)
                    acc = tuple(
                        jnp.where(is_new, r[k], acc[k] + r[k])
                        for k in range(8))
                    prev = n
                return ((prev, ord_, id0), acc)

            return lax.fori_loop(0, trip // 16, grp_body,
                                 ((prev, ord_, id0), acc))

        def pair_body(m, ec):
            jw0 = 2 * m
            prefetch(jw0 + 1, BUFS[1])
            ec = process(jw0, ec, BUFS[0])
            prefetch(jw0 + 2, BUFS[0])
            ec = process(jw0 + 1, ec, BUFS[1])
            return ec

        zero8 = tuple(jnp.zeros((16,), jnp.float32) for _ in range(8))
        prefetch(0, BUFS[0])
        npairs = (nwin + 1) // 2
        carry = lax.fori_loop(
            0, npairs, pair_body,
            ((jnp.int32(-1), jnp.int32(0), jnp.int32(-1)), zero8))
        pltpu.make_async_copy(feats_h.at[idxv0], rowsv0, gsem0).wait()
        (prev, ord_, id0) = carry[0]
        acc = carry[1]

        for k in range(8):
            ringv[pl.ds(k * 16, 16)] = acc[k]
        def carry_first(_):
            pltpu.async_copy(ringv.at[pl.ds(0, 128)],
                             brows_h.at[pl.ds((2 * wid) * 128, 128)],
                             msem).wait()
            return 0
        def carry_second(_):
            pltpu.async_copy(ringv.at[pl.ds(0, 128)],
                             brows_h.at[pl.ds((2 * wid + 1) * 128, 128)],
                             msem).wait()
            return 0
        _ = lax.cond(ord_ == 0, carry_first, carry_second, 0)
        id0f = jnp.where(ord_ == 0, prev, id0)
        id1f = jnp.where(ord_ == 0, jnp.int32(-1), prev)

        lane = lax.iota(jnp.int32, 16)
        idsv[...] = jnp.where(lane == 0, id0f,
                              jnp.where(lane == 1, id1f, jnp.int32(-1)))
        pltpu.sync_copy(idsv, bids_h.at[wid])

        for s in range(4):
            def dr(_):
                pltpu.make_async_copy(
                    ringv.at[pl.ds(s * 128, 128)],
                    agg_h.at[pl.ds(0, 128)], fsems[s]).wait()
                return 0
            _ = lax.cond(ord_ > s, dr, lambda _: 0, 0)

        def nf_read(_):
            pltpu.sync_copy(dsts_h.at[pl.ds(b_hi, 16)], nfv)
            return nfv[...][0]
        next_first = lax.cond(wid == 31, lambda _: jnp.int32(NPAD - 1),
                              nf_read, 0)
        zlo = prev
        zhi = next_first
        def zrow(j, _):
            pltpu.sync_copy(zerov, agg_h.at[pl.ds((zlo + j) * 128, 128)])
            return 0
        _ = lax.fori_loop(0, zhi - zlo + 1, zrow, 0)
        def w0_extra(_):
            pltpu.sync_copy(dsts_h.at[pl.ds(0, 16)], nfv)
            first0 = nfv[...][0]
            def zr(j, _):
                pltpu.sync_copy(zerov, agg_h.at[pl.ds(j * 128, 128)])
                return 0
            _ = lax.fori_loop(0, first0 + 1, zr, 0)
            return 0
        _ = lax.cond(wid == 0, w0_extra, lambda _: 0, 0)

    return scat(feats, dsts_pad, srcs_pad)


def _strip_reduce128(ref, fn, sizes):
    total = jnp.zeros((1, 128), jnp.float32)
    pos = 0
    for sz in sizes:
        base = pos
        def body(i, acc):
            return acc + fn(ref[pl.ds((base + i) * 8, 8), :])
        acc = lax.fori_loop(0, sz, body, jnp.zeros((8, 128), jnp.float32))
        pos += sz
        a4 = acc[:4] + acc[4:]
        a2 = a4[:2] + a4[2:]
        total = total + (a2[0:1] + a2[1:2])
    return total


def _lane_reduce40(ref, fn):
    def body(i, acc):
        return acc + fn(ref[pl.ds(i * 128, 128), :])
    acc = lax.fori_loop(0, 78, body, jnp.zeros((128, C), jnp.float32))
    tailblk = fn(ref[pl.ds(9984, 16), :])
    acc = acc + jnp.pad(tailblk, ((0, 112), (0, 0)))
    b = acc[0:8]
    for g in range(1, 16):
        b = b + acc[8 * g:8 * g + 8]
    b4 = b[:4] + b[4:]
    b2 = b4[:2] + b4[2:]
    return b2[0:1] + b2[1:2]


def _merge_boundary(aggm_ref, bids_ref, brows_ref):
    iota8 = lax.broadcasted_iota(jnp.int32, (8, 128), 0)
    def body(e, _):
        w = e // 2
        k = e - 2 * w
        nid = bids_ref[w, k]
        valid = nid >= 0
        nidc = jnp.where(valid, nid, 0)
        n8 = pl.multiple_of((nidc // 8) * 8, 8)
        roff = nidc - n8
        block = aggm_ref[pl.ds(n8, 8), :]
        row = brows_ref[pl.ds(e, 1), :]
        rowb = jnp.broadcast_to(row, (8, 128))
        mask = (iota8 == roff) & valid
        aggm_ref[pl.ds(n8, 8), :] = jnp.where(mask, block + rowb, block)
        return 0
    lax.fori_loop(0, 64, body, 0)


def _bn_chain128(z_ref, g, b):
    m = _strip_reduce128(z_ref, lambda u: u, (1250,)) * np.float32(1e-4)
    v = _strip_reduce128(z_ref, lambda u: (u - m) * (u - m),
                         (625, 625)) * np.float32(1e-4)
    sd = jnp.sqrt(v + np.float32(1e-5))
    return (z_ref[...] - m) / sd * g + b


def _tc_layer1(x, agg, bids, brows, W1a, b1a, g1a, be1a, W1b, b1b, g1b, be1b):
    def kern(x_ref, agg_ref, bids_ref, brows_ref,
             w1a_ref, b1a_ref, g1a_ref, be1a_ref,
             w1b_ref, b1b_ref, g1b_ref, be1b_ref,
             out_ref, aggm_ref, z_ref, t_ref):
        aggm_ref[...] = agg_ref[...]
        _merge_boundary(aggm_ref, bids_ref, brows_ref)
        t_ref[...] = x_ref[...] + aggm_ref[...]
        z_ref[...] = jnp.dot(t_ref[...], w1a_ref[...],
                             preferred_element_type=jnp.float32) + b1a_ref[...]
        h = _bn_chain128(z_ref, g1a_ref[...], be1a_ref[...])
        t_ref[...] = jnp.maximum(h, 0.0)
        z_ref[...] = jnp.dot(t_ref[...], w1b_ref[...],
                             preferred_element_type=jnp.float32) + b1b_ref[...]
        h2 = _bn_chain128(z_ref, g1b_ref[...], be1b_ref[...])
        out_ref[...] = jnp.maximum(h2, 0.0)

    vspec = pl.BlockSpec(memory_space=pltpu.VMEM)
    specs = [pl.BlockSpec(memory_space=pltpu.SMEM) if i == 2 else vspec
             for i in range(12)]
    return pl.pallas_call(
        kern,
        out_shape=jax.ShapeDtypeStruct((N, 128), jnp.float32),
        in_specs=specs,
        scratch_shapes=[
            pltpu.VMEM((N, 128), jnp.float32),
            pltpu.VMEM((N, 128), jnp.float32),
            pltpu.VMEM((N, 128), jnp.float32),
        ],
    )(x, agg, bids, brows,
      W1a, b1a.reshape(1, H), g1a.reshape(1, H), be1a.reshape(1, H),
      W1b, b1b.reshape(1, H), g1b.reshape(1, H), be1b.reshape(1, H))


def _tc_layer2(h1, agg, bids, brows, W2a, b2a, g2a, be2a, W2b, b2b, g2b, be2b):
    def kern(x_ref, agg_ref, bids_ref, brows_ref,
             w2a_ref, b2a_ref, g2a_ref, be2a_ref,
             w2b_ref, b2b_ref, g2b_ref, be2b_ref,
             out_ref, aggm_ref, z_ref, t_ref, z40_ref):
        aggm_ref[...] = agg_ref[...]
        _merge_boundary(aggm_ref, bids_ref, brows_ref)
        t_ref[...] = x_ref[...] + aggm_ref[...]
        z_ref[...] = jnp.dot(t_ref[...], w2a_ref[...],
                             preferred_element_type=jnp.float32) + b2a_ref[...]
        h = _bn_chain128(z_ref, g2a_ref[...], be2a_ref[...])
        t_ref[...] = jnp.maximum(h, 0.0)
        z40_ref[...] = jnp.dot(t_ref[...], w2b_ref[...],
                               preferred_element_type=jnp.float32) + b2b_ref[...]
        m40 = _lane_reduce40(z40_ref, lambda u: u) * np.float32(1e-4)
        v40 = _lane_reduce40(z40_ref, lambda u: (u - m40) * (u - m40)) \
            * np.float32(1e-4)
        sd40 = jnp.sqrt(v40 + np.float32(1e-5))
        g40 = g2b_ref[...]
        be40 = be2b_ref[...]
        fin = _lane_reduce40(
            z40_ref, lambda u: (u - m40) / sd40 * g40 + be40) * np.float32(1e-4)
        out_ref[...] = fin

    vspec = pl.BlockSpec(memory_space=pltpu.VMEM)
    specs = [pl.BlockSpec(memory_space=pltpu.SMEM) if i == 2 else vspec
             for i in range(12)]
    return pl.pallas_call(
        kern,
        out_shape=jax.ShapeDtypeStruct((1, C), jnp.float32),
        in_specs=specs,
        scratch_shapes=[
            pltpu.VMEM((N, 128), jnp.float32),
            pltpu.VMEM((N, 128), jnp.float32),
            pltpu.VMEM((N, 128), jnp.float32),
            pltpu.VMEM((N, C), jnp.float32),
        ],
    )(h1, agg, bids, brows,
      W2a, b2a.reshape(1, H), g2a.reshape(1, H), be2a.reshape(1, H),
      W2b, b2b.reshape(1, C), g2b.reshape(1, C), be2b.reshape(1, C))


def kernel(in_feat, edge_index, W1a, b1a, g1a, be1a, W1b, b1b, g1b, be1b,
           W2a, b2a, g2a, be2a, W2b, b2b, g2b, be2b):
    src = edge_index[0]
    dst = edge_index[1]
    dst_s, src_s = lax.sort((dst, src), dimension=0, is_stable=True,
                            num_keys=1)
    pad = jnp.zeros((EPAD - E,), jnp.int32)
    dsts_pad = jnp.concatenate([dst_s, pad])
    srcs_pad = jnp.concatenate([src_s, pad])

    agg1d, bids, brows1d = _sc_segment_sum(in_feat, dsts_pad, srcs_pad)
    agg = agg1d.reshape(NPAD, 128)[:N]
    brows = brows1d.reshape(64, 128)
    h1 = _tc_layer1(in_feat, agg, bids, brows,
                    W1a, b1a, g1a, be1a, W1b, b1b, g1b, be1b)

    agg1d2, bids2, brows1d2 = _sc_segment_sum(h1, dsts_pad, srcs_pad)
    agg2 = agg1d2.reshape(NPAD, 128)[:N]
    brows2 = brows1d2.reshape(64, 128)
    out = _tc_layer2(h1, agg2, bids2, brows2,
                     W2a, b2a, g2a, be2a, W2b, b2b, g2b, be2b)
    return out
